# baseline (speedup 1.0000x reference)
import functools

import jax
import jax.numpy as jnp
from jax import lax
from jax.experimental import pallas as pl
from jax.experimental.pallas import tpu as pltpu
from jax.experimental.pallas import tpu_sc as plsc

N = 10000
E = 320000
D = 128
A = 256
HOUT = 64

NC = 2
NS = 16
NW = NC * NS
BE = 128
KBUF = 2
EP = 327680
TBLK = 160
CHB = 16
NCH = TBLK // CHB
NBLKC = 80

NT = 10240
NH = NT // NC
NHP = NH + 128
LDUMMY = NH
RPTH = NHP // NS
PCAP = (TBLK + 2 * KBUF + 2) * BE

DUMMY = A
ASL = 384
RPT2 = ASL // NS
APT = A // NS

_MESH = dict(core_axis_name="c", subcore_axis_name="s")


@functools.partial(
    pl.kernel,
    out_type=jax.ShapeDtypeStruct((NC, NHP, D), jnp.float32),
    mesh=plsc.VectorSubcoreMesh(**_MESH),
    compiler_params=pltpu.CompilerParams(needs_layout_passes=False),
    scratch_types=[
        pltpu.VMEM((CHB, BE), jnp.int32),
        pltpu.VMEM((CHB, BE), jnp.int32),
        pltpu.VMEM((PCAP + 16,), jnp.int32),
        pltpu.VMEM((PCAP + 16,), jnp.int32),
        pltpu.VMEM((1, BE), jnp.int32),
        [pltpu.VMEM((BE, D), jnp.float32) for _ in range(KBUF)],
        pltpu.VMEM_SHARED((NHP, D), jnp.float32),
        [pltpu.SemaphoreType.DMA for _ in range(KBUF)],
    ],
)
def _segsum_kernel(table, srcs, dsts, zeros, out,
                   ssrc, sdst, psrc, pdst, idx2d, bufs, acc, sems):
    cid = lax.axis_index("c")
    sid = lax.axis_index("s")
    lo = cid * NH
    pltpu.sync_copy(zeros, acc.at[pl.ds(sid * RPTH, RPTH)])

    def chunk(c, cnt):
        pltpu.sync_copy(srcs.at[sid, pl.ds(c * CHB, CHB)], ssrc)
        pltpu.sync_copy(dsts.at[sid, pl.ds(c * CHB, CHB)], sdst)

        def blk(j, cnt):
            for v in range(BE // 16):
                d16 = sdst[j, pl.ds(v * 16, 16)]
                s16 = ssrc[j, pl.ds(v * 16, 16)]
                l16 = d16 - lo
                m = (l16 >= 0) & (l16 < NH)
                mi = m.astype(jnp.int32)
                pos = cnt + plsc.cumsum(mi) - mi
                plsc.store_scatter(psrc, [pos], s16, mask=m)
                plsc.store_scatter(pdst, [pos], l16, mask=m)
                cnt = cnt + jnp.sum(mi)
            return cnt

        return lax.fori_loop(0, CHB, blk, cnt)

    cnt = lax.fori_loop(0, NCH, chunk, jnp.int32(0))

    zero16 = jnp.zeros((16,), jnp.int32)
    ld16 = jnp.full((16,), LDUMMY, jnp.int32)
    for b in range(2 * KBUF + 2):
        for v in range(BE // 16):
            psrc[pl.ds(cnt + b * BE + v * 16, 16)] = zero16
            pdst[pl.ds(cnt + b * BE + v * 16, 16)] = ld16

    nblk = (cnt + BE - 1) // BE
    trips = (nblk + KBUF - 1) // KBUF
    for t in range(KBUF):
        pltpu.async_copy(table.at[psrc.at[pl.ds(t * BE, BE)]], bufs[t], sems[t])

    def body(k, carry):
        for t in range(KBUF):
            j = k * KBUF + t
            pltpu.make_async_copy(
                table.at[psrc.at[pl.ds(j * BE, BE)]], bufs[t], sems[t]).wait()
            for v in range(BE // 16):
                idx2d[0, pl.ds(v * 16, 16)] = pdst[pl.ds(j * BE + v * 16, 16)]
            pltpu.sync_copy(bufs[t], acc.at[idx2d.at[0]], add=True)
            pltpu.async_copy(
                table.at[psrc.at[pl.ds((j + KBUF) * BE, BE)]], bufs[t], sems[t])
        return carry

    lax.fori_loop(0, trips, body, 0)
    for t in range(KBUF):
        pltpu.make_async_copy(
            table.at[psrc.at[pl.ds(0, BE)]], bufs[t], sems[t]).wait()
    plsc.subcore_barrier()
    pltpu.sync_copy(acc.at[pl.ds(sid * RPTH, RPTH)],
                    out.at[cid, pl.ds(sid * RPTH, RPTH)])


@functools.partial(
    pl.kernel,
    out_type=jax.ShapeDtypeStruct((3, A, D), jnp.float32),
    mesh=plsc.VectorSubcoreMesh(**_MESH),
    compiler_params=pltpu.CompilerParams(needs_layout_passes=False),
    scratch_types=[
        pltpu.VMEM((NBLKC, BE), jnp.int32),
        pltpu.VMEM((NBLKC, BE), jnp.int32),
        pltpu.VMEM((NT,), jnp.int32),
        pltpu.VMEM((A,), jnp.int32),
        pltpu.VMEM((APT,), jnp.int32),
        pltpu.VMEM((NBLKC * BE + 2 * BE,), jnp.int32),
        pltpu.VMEM((NBLKC * BE + 2 * BE,), jnp.int32),
        pltpu.VMEM((1, BE), jnp.int32),
        pltpu.VMEM((BE, D), jnp.float32),
        pltpu.VMEM((APT, D), jnp.float32),
        pltpu.VMEM_SHARED((ASL, D), jnp.float32),
        pltpu.SemaphoreType.DMA,
    ],
)
def _agent_conv_kernel(h, srcs, dsts, aidx, zeros, dslot, out,
                       src_v, dst_v, slot, aidx_v, aslot_v, csrc, cslot,
                       idx2d, rows, arow, acc, sem):
    cid = lax.axis_index("c")
    sid = lax.axis_index("s")
    wid = sid * NC + cid
    pltpu.sync_copy(zeros.at[pl.ds(0, RPT2)], acc.at[pl.ds(sid * RPT2, RPT2)])
    pltpu.sync_copy(dslot, slot)
    pltpu.sync_copy(aidx, aidx_v)
    for v in range(A // 16):
        a16 = aidx_v[pl.ds(v * 16, 16)]
        s16 = lax.iota(jnp.int32, 16) + v * 16
        plsc.store_scatter(slot, [a16], s16)
    pltpu.sync_copy(srcs.at[wid], src_v)
    pltpu.sync_copy(dsts.at[wid], dst_v)

    def cblk(j, cnt):
        for v in range(BE // 16):
            d16 = dst_v[j, pl.ds(v * 16, 16)]
            s16 = src_v[j, pl.ds(v * 16, 16)]
            f16 = plsc.load_gather(slot, [d16])
            m = f16 < DUMMY
            mi = m.astype(jnp.int32)
            pos = cnt + plsc.cumsum(mi) - mi
            plsc.store_scatter(csrc, [pos], s16, mask=m)
            plsc.store_scatter(cslot, [pos], f16, mask=m)
            cnt = cnt + jnp.sum(mi)
        return cnt

    cnt = lax.fori_loop(0, NBLKC, cblk, jnp.int32(0))
    zero16 = jnp.zeros((16,), jnp.int32)
    dummy16 = jnp.full((16,), DUMMY, jnp.int32)
    for v in range(BE // 16):
        csrc[pl.ds(cnt + v * 16, 16)] = zero16
        cslot[pl.ds(cnt + v * 16, 16)] = dummy16

    def gblk(b, carry):
        for v in range(BE // 16):
            idx2d[0, pl.ds(v * 16, 16)] = cslot[pl.ds(b * BE + v * 16, 16)]
        pltpu.async_copy(h.at[csrc.at[pl.ds(b * BE, BE)]], rows, sem).wait()
        pltpu.sync_copy(rows, acc.at[idx2d.at[0]], add=True)
        return carry

    nblk = (cnt + BE - 1) // BE
    lax.fori_loop(0, nblk, gblk, 0)
    plsc.subcore_barrier()

    a16 = aidx_v[pl.ds(sid * APT, 16)]
    aslot_v[pl.ds(0, 16)] = plsc.load_gather(slot, [a16])
    pltpu.async_copy(acc.at[aslot_v], arow, sem).wait()
    pltpu.sync_copy(arow, out.at[cid, pl.ds(sid * APT, APT)])

    @pl.when(cid == 0)
    def _():
        pltpu.async_copy(h.at[aidx_v.at[pl.ds(sid * APT, APT)]], arow, sem).wait()
        pltpu.sync_copy(arow, out.at[2, pl.ds(sid * APT, APT)])


def _dot_t(a, w):
    return lax.dot_general(a, w, (((1,), (1,)), ((), ())),
                           preferred_element_type=jnp.float32)


RB = 1024


def _tc_linear(halves, x, w_rel, b_rel, w_root):
    def body(p_ref, x_ref, wr_ref, br_ref, wo_ref, o_ref):
        agg = p_ref[0]
        y = _dot_t(agg, wr_ref[...]) + br_ref[...] + _dot_t(x_ref[...], wo_ref[...])
        o_ref[...] = jnp.maximum(y, 0.0)

    nb = NT // RB
    per_half = NH // RB
    return pl.pallas_call(
        body,
        grid=(nb,),
        in_specs=[
            pl.BlockSpec((1, RB, D), lambda i: (i // per_half, i % per_half, 0)),
            pl.BlockSpec((RB, D), lambda i: (i, 0)),
            pl.BlockSpec((D, D), lambda i: (0, 0)),
            pl.BlockSpec((1, D), lambda i: (0, 0)),
            pl.BlockSpec((D, D), lambda i: (0, 0)),
        ],
        out_specs=pl.BlockSpec((RB, D), lambda i: (i, 0)),
        out_shape=jax.ShapeDtypeStruct((NT, D), jnp.float32),
    )(halves, x, w_rel, b_rel, w_root)


def _tc_head(sel3, w_rel2, b_rel2, w_root2, wp1, bp1, wp2, bp2, wp3, bp3):
    def body(s_ref, wr, br, wo, w1, b1, w2, b2, w3, b3, o_ref):
        agg = s_ref[0] + s_ref[1]
        emb = jnp.maximum(_dot_t(agg, wr[...]) + br[...] + _dot_t(s_ref[2], wo[...]), 0.0)
        t = jnp.maximum(_dot_t(emb, w1[...]) + b1[...], 0.0)
        t = jnp.maximum(_dot_t(t, w2[...]) + b2[...], 0.0)
        o_ref[...] = _dot_t(t, w3[...]) + b3[...]

    return pl.pallas_call(
        body,
        out_shape=jax.ShapeDtypeStruct((A, HOUT), jnp.float32),
    )(sel3, w_rel2, b_rel2, w_root2, wp1, bp1, wp2, bp2, wp3, bp3)


def kernel(node_features, edge_index, agent_idx,
           W_rel1, b_rel1, W_root1,
           W_rel2, b_rel2, W_root2,
           Wp1, bp1, Wp2, bp2, Wp3, bp3):
    pad = EP - E
    src_f = jnp.concatenate([edge_index[0], jnp.zeros((pad,), jnp.int32)])
    dst_f = jnp.concatenate([edge_index[1], jnp.full((pad,), N, jnp.int32)])
    src_a = src_f.reshape(NS, TBLK, BE)
    dst_a = dst_f.reshape(NS, TBLK, BE)
    src_c = src_f.reshape(NW, NBLKC, BE)
    dst_c = dst_f.reshape(NW, NBLKC, BE)
    zeros = jnp.zeros((RPTH, D), jnp.float32)
    dslot = jnp.full((NT,), DUMMY, jnp.int32)
    xp = jnp.pad(node_features, ((0, NT - N), (0, 0)))

    p1 = _segsum_kernel(node_features, src_a, dst_a, zeros)
    h = _tc_linear(p1[:, :NH, :], xp, W_rel1, b_rel1.reshape(1, D), W_root1)
    sel3 = _agent_conv_kernel(h, src_c, dst_c, agent_idx, zeros, dslot)
    out = _tc_head(sel3, W_rel2, b_rel2.reshape(1, D), W_root2,
                   Wp1, bp1.reshape(1, D), Wp2, bp2.reshape(1, D),
                   Wp3, bp3.reshape(1, HOUT))
    return out.reshape(A, 16, 4)


# speedup vs baseline: 1.2118x; 1.2118x over previous
import functools

import jax
import jax.numpy as jnp
from jax import lax
from jax.experimental import pallas as pl
from jax.experimental.pallas import tpu as pltpu
from jax.experimental.pallas import tpu_sc as plsc

N = 10000
E = 320000
D = 128
A = 256
HOUT = 64

NC = 2
NS = 16
NW = NC * NS
BE = 128
KBUF = 2
EP = 327680
EPW = EP // NW
NBLKC = EPW // BE
CHB = 16
NCHW = NBLKC // CHB

NT = 10240
RPZ = NT // NS
ZR = 32

DUMMY = A
ASL = 384
RPT2 = ASL // NS
APT = A // NS

_MESH = dict(core_axis_name="c", subcore_axis_name="s")


@functools.partial(
    pl.kernel,
    out_type=jax.ShapeDtypeStruct((NC, NT, D), jnp.float32),
    mesh=plsc.VectorSubcoreMesh(**_MESH),
    compiler_params=pltpu.CompilerParams(needs_layout_passes=False),
    scratch_types=[
        pltpu.VMEM((EPW + KBUF * BE,), jnp.int32),
        pltpu.VMEM((CHB * BE,), jnp.int32),
        pltpu.VMEM((16, D), jnp.float32),
        [pltpu.VMEM((BE, D), jnp.float32) for _ in range(KBUF)],
        pltpu.VMEM_SHARED((NT, D), jnp.float32),
        [pltpu.SemaphoreType.DMA for _ in range(KBUF)],
    ],
)
def _segsum_kernel(table, srcs, dsts, zeros, out, vsrc, vdst, vzero, bufs, acc, sems):
    cid = lax.axis_index("c")
    sid = lax.axis_index("s")
    wid = sid * NC + cid
    pltpu.sync_copy(zeros.at[pl.ds(0, 16)], vzero)
    for r in range(RPZ // 16):
        pltpu.sync_copy(vzero, acc.at[pl.ds(sid * RPZ + r * 16, 16)])
    pltpu.sync_copy(srcs.at[wid], vsrc.at[pl.ds(0, EPW)])
    zero16 = jnp.zeros((16,), jnp.int32)
    for v in range(KBUF * BE // 16):
        vsrc[pl.ds(EPW + v * 16, 16)] = zero16
    plsc.subcore_barrier()

    for t in range(KBUF):
        pltpu.async_copy(table.at[vsrc.at[pl.ds(t * BE, BE)]], bufs[t], sems[t])

    def chunk(c, carry):
        pltpu.sync_copy(dsts.at[wid, pl.ds(c * CHB * BE, CHB * BE)], vdst)
        for b in range(CHB):
            t = b % KBUF
            j = c * CHB + b
            pltpu.make_async_copy(
                table.at[vsrc.at[pl.ds(j * BE, BE)]], bufs[t], sems[t]).wait()
            pltpu.sync_copy(bufs[t], acc.at[vdst.at[pl.ds(b * BE, BE)]], add=True)
            pltpu.async_copy(
                table.at[vsrc.at[pl.ds((j + KBUF) * BE, BE)]], bufs[t], sems[t])
        return carry

    lax.fori_loop(0, NCHW, chunk, 0)
    for t in range(KBUF):
        pltpu.make_async_copy(
            table.at[vsrc.at[pl.ds(EPW + t * BE, BE)]], bufs[t], sems[t]).wait()
    plsc.subcore_barrier()
    pltpu.sync_copy(acc.at[pl.ds(sid * RPZ, RPZ)],
                    out.at[cid, pl.ds(sid * RPZ, RPZ)])


@functools.partial(
    pl.kernel,
    out_type=jax.ShapeDtypeStruct((3, A, D), jnp.float32),
    mesh=plsc.VectorSubcoreMesh(**_MESH),
    compiler_params=pltpu.CompilerParams(needs_layout_passes=False),
    scratch_types=[
        pltpu.VMEM((NBLKC, BE), jnp.int32),
        pltpu.VMEM((NBLKC, BE), jnp.int32),
        pltpu.VMEM((NT,), jnp.int32),
        pltpu.VMEM((A,), jnp.int32),
        pltpu.VMEM((APT,), jnp.int32),
        pltpu.VMEM((NBLKC * BE + 2 * BE,), jnp.int32),
        pltpu.VMEM((NBLKC * BE + 2 * BE,), jnp.int32),
        pltpu.VMEM((1, BE), jnp.int32),
        pltpu.VMEM((BE, D), jnp.float32),
        pltpu.VMEM((APT, D), jnp.float32),
        pltpu.VMEM_SHARED((ASL, D), jnp.float32),
        pltpu.SemaphoreType.DMA,
    ],
)
def _agent_conv_kernel(h, srcs, dsts, aidx, zeros, dslot, out,
                       src_v, dst_v, slot, aidx_v, aslot_v, csrc, cslot,
                       idx2d, rows, arow, acc, sem):
    cid = lax.axis_index("c")
    sid = lax.axis_index("s")
    wid = sid * NC + cid
    pltpu.sync_copy(zeros.at[pl.ds(0, RPT2)], acc.at[pl.ds(sid * RPT2, RPT2)])
    pltpu.sync_copy(dslot, slot)
    pltpu.sync_copy(aidx, aidx_v)
    for v in range(A // 16):
        a16 = aidx_v[pl.ds(v * 16, 16)]
        s16 = lax.iota(jnp.int32, 16) + v * 16
        plsc.store_scatter(slot, [a16], s16)
    pltpu.sync_copy(srcs.at[wid], src_v)
    pltpu.sync_copy(dsts.at[wid], dst_v)

    def cblk(j, cnt):
        for v in range(BE // 16):
            d16 = dst_v[j, pl.ds(v * 16, 16)]
            s16 = src_v[j, pl.ds(v * 16, 16)]
            f16 = plsc.load_gather(slot, [d16])
            m = f16 < DUMMY
            mi = m.astype(jnp.int32)
            pos = cnt + plsc.cumsum(mi) - mi
            plsc.store_scatter(csrc, [pos], s16, mask=m)
            plsc.store_scatter(cslot, [pos], f16, mask=m)
            cnt = cnt + jnp.sum(mi)
        return cnt

    cnt = lax.fori_loop(0, NBLKC, cblk, jnp.int32(0))
    zero16 = jnp.zeros((16,), jnp.int32)
    dummy16 = jnp.full((16,), DUMMY, jnp.int32)
    for v in range(BE // 16):
        csrc[pl.ds(cnt + v * 16, 16)] = zero16
        cslot[pl.ds(cnt + v * 16, 16)] = dummy16

    def gblk(b, carry):
        for v in range(BE // 16):
            idx2d[0, pl.ds(v * 16, 16)] = cslot[pl.ds(b * BE + v * 16, 16)]
        pltpu.async_copy(h.at[csrc.at[pl.ds(b * BE, BE)]], rows, sem).wait()
        pltpu.sync_copy(rows, acc.at[idx2d.at[0]], add=True)
        return carry

    nblk = (cnt + BE - 1) // BE
    lax.fori_loop(0, nblk, gblk, 0)
    plsc.subcore_barrier()

    a16 = aidx_v[pl.ds(sid * APT, 16)]
    aslot_v[pl.ds(0, 16)] = plsc.load_gather(slot, [a16])
    pltpu.async_copy(acc.at[aslot_v], arow, sem).wait()
    pltpu.sync_copy(arow, out.at[cid, pl.ds(sid * APT, APT)])

    @pl.when(cid == 0)
    def _():
        pltpu.async_copy(h.at[aidx_v.at[pl.ds(sid * APT, APT)]], arow, sem).wait()
        pltpu.sync_copy(arow, out.at[2, pl.ds(sid * APT, APT)])


def _dot_t(a, w):
    return lax.dot_general(a, w, (((1,), (1,)), ((), ())),
                           preferred_element_type=jnp.float32)


RB = 1024


def _tc_linear(partials, x, w_rel, b_rel, w_root):
    def body(p_ref, x_ref, wr_ref, br_ref, wo_ref, o_ref):
        agg = p_ref[0] + p_ref[1]
        y = _dot_t(agg, wr_ref[...]) + br_ref[...] + _dot_t(x_ref[...], wo_ref[...])
        o_ref[...] = jnp.maximum(y, 0.0)

    nb = NT // RB
    return pl.pallas_call(
        body,
        grid=(nb,),
        in_specs=[
            pl.BlockSpec((NC, RB, D), lambda i: (0, i, 0)),
            pl.BlockSpec((RB, D), lambda i: (i, 0)),
            pl.BlockSpec((D, D), lambda i: (0, 0)),
            pl.BlockSpec((1, D), lambda i: (0, 0)),
            pl.BlockSpec((D, D), lambda i: (0, 0)),
        ],
        out_specs=pl.BlockSpec((RB, D), lambda i: (i, 0)),
        out_shape=jax.ShapeDtypeStruct((NT, D), jnp.float32),
    )(partials, x, w_rel, b_rel, w_root)


def _tc_head(sel3, w_rel2, b_rel2, w_root2, wp1, bp1, wp2, bp2, wp3, bp3):
    def body(s_ref, wr, br, wo, w1, b1, w2, b2, w3, b3, o_ref):
        agg = s_ref[0] + s_ref[1]
        emb = jnp.maximum(_dot_t(agg, wr[...]) + br[...] + _dot_t(s_ref[2], wo[...]), 0.0)
        t = jnp.maximum(_dot_t(emb, w1[...]) + b1[...], 0.0)
        t = jnp.maximum(_dot_t(t, w2[...]) + b2[...], 0.0)
        o_ref[...] = _dot_t(t, w3[...]) + b3[...]

    return pl.pallas_call(
        body,
        out_shape=jax.ShapeDtypeStruct((A, HOUT), jnp.float32),
    )(sel3, w_rel2, b_rel2, w_root2, wp1, bp1, wp2, bp2, wp3, bp3)


def kernel(node_features, edge_index, agent_idx,
           W_rel1, b_rel1, W_root1,
           W_rel2, b_rel2, W_root2,
           Wp1, bp1, Wp2, bp2, Wp3, bp3):
    pad = EP - E
    src_f = jnp.concatenate([edge_index[0], jnp.zeros((pad,), jnp.int32)])
    dst_f = jnp.concatenate([edge_index[1], jnp.full((pad,), N, jnp.int32)])
    src_w = src_f.reshape(NW, EPW)
    dst_w = dst_f.reshape(NW, EPW)
    src_c = src_f.reshape(NW, NBLKC, BE)
    dst_c = dst_f.reshape(NW, NBLKC, BE)
    zeros = jnp.zeros((ZR, D), jnp.float32)
    dslot = jnp.full((NT,), DUMMY, jnp.int32)
    xp = jnp.pad(node_features, ((0, NT - N), (0, 0)))

    p1 = _segsum_kernel(node_features, src_w, dst_w, zeros)
    h = _tc_linear(p1, xp, W_rel1, b_rel1.reshape(1, D), W_root1)
    sel3 = _agent_conv_kernel(h, src_c, dst_c, agent_idx, zeros, dslot)
    out = _tc_head(sel3, W_rel2, b_rel2.reshape(1, D), W_root2,
                   Wp1, bp1.reshape(1, D), Wp2, bp2.reshape(1, D),
                   Wp3, bp3.reshape(1, HOUT))
    return out.reshape(A, 16, 4)
